# Initial kernel scaffold; baseline (speedup 1.0000x reference)
import functools

import jax
import jax.numpy as jnp
from jax import lax
from jax.experimental import pallas as pl
from jax.experimental.pallas import tpu as pltpu
from jax.experimental.pallas import tpu_sc as plsc

_L = 128
_CHUNK = 8
_D = 32


def _emb_kernel(n_rows_total, table_hbm, idx_hbm, out_hbm, idx_v, rows_v, sem):
    nc = 2
    wid = lax.axis_index("s") * nc + lax.axis_index("c")
    nw = 32
    rows_per_w = n_rows_total // nw
    base = wid * rows_per_w
    pltpu.sync_copy(idx_hbm.at[pl.ds(base, rows_per_w)], idx_v)

    n_chunks = rows_per_w // _CHUNK

    def body(c, carry):
        row0 = c * _CHUNK
        copies = []
        for j in range(_CHUNK):
            copies.append(
                pltpu.async_copy(
                    table_hbm.at[idx_v.at[row0 + j]], rows_v.at[j], sem
                )
            )
        for cp in copies:
            cp.wait()
        pltpu.sync_copy(rows_v, out_hbm.at[pl.ds(base + row0, _CHUNK)])
        return carry

    lax.fori_loop(0, n_chunks, body, 0)


def kernel(x, emb_weight):
    n, m = x.shape
    total = n * m
    n_rows = total // _L
    idx2d = x.reshape(n_rows, _L).astype(jnp.int32)

    mesh = plsc.VectorSubcoreMesh(core_axis_name="c", subcore_axis_name="s")
    rows_per_w = n_rows // 32

    k = functools.partial(
        pl.kernel,
        mesh=mesh,
        out_type=jax.ShapeDtypeStruct((n_rows, _L, _D), jnp.float32),
        scratch_types=[
            pltpu.VMEM((rows_per_w, _L), jnp.int32),
            pltpu.VMEM((_CHUNK, _L, _D), jnp.float32),
            pltpu.SemaphoreType.DMA,
        ],
    )(functools.partial(_emb_kernel, n_rows))

    out = k(emb_weight, idx2d)
    return out.reshape(n, m, _D)


# speedup vs baseline: 1.5591x; 1.5591x over previous
import functools

import jax
import jax.numpy as jnp
from jax import lax
from jax.experimental import pallas as pl
from jax.experimental.pallas import tpu as pltpu
from jax.experimental.pallas import tpu_sc as plsc

_L = 128
_CHUNK = 8
_D = 32


def _emb_kernel(n_rows_total, table_hbm, idx_hbm, out_hbm, idx_v, rows_v, sem):
    nc = 2
    wid = lax.axis_index("s") * nc + lax.axis_index("c")
    nw = 32
    rows_per_w = n_rows_total // nw
    base = wid * rows_per_w
    pltpu.sync_copy(idx_hbm.at[pl.ds(base, rows_per_w)], idx_v)

    n_chunks = rows_per_w // _CHUNK

    def body(c, carry):
        row0 = c * _CHUNK
        copies = []
        for j in range(_CHUNK):
            copies.append(
                pltpu.async_copy(
                    table_hbm.at[idx_v.at[row0 + j]], rows_v.at[j], sem
                )
            )
        for cp in copies:
            cp.wait()
        pltpu.sync_copy(rows_v, out_hbm.at[pl.ds(base + row0, _CHUNK)])
        return carry

    lax.fori_loop(0, n_chunks, body, 0)


def kernel(x, emb_weight):
    n, m = x.shape
    total = n * m
    n_rows = total // _L
    idx2d = x.reshape(n_rows, _L).astype(jnp.int32)

    mesh = plsc.VectorSubcoreMesh(core_axis_name="c", subcore_axis_name="s")
    rows_per_w = n_rows // 32

    k = functools.partial(
        pl.kernel,
        mesh=mesh,
        out_type=jax.ShapeDtypeStruct((n_rows, _L, _D), jnp.float32),
        scratch_types=[
            pltpu.VMEM((rows_per_w, _L), jnp.int32),
            pltpu.VMEM((_CHUNK, _L, _D), jnp.float32),
            pltpu.SemaphoreType.DMA,
        ],
        compiler_params=pltpu.CompilerParams(use_tc_tiling_on_sc=False),
    )(functools.partial(_emb_kernel, n_rows))

    out = k(emb_weight, idx2d)
    return out.reshape(n, m, _D)


# speedup vs baseline: 1.5691x; 1.0064x over previous
import functools

import jax
import jax.numpy as jnp
from jax import lax
from jax.experimental import pallas as pl
from jax.experimental.pallas import tpu as pltpu
from jax.experimental.pallas import tpu_sc as plsc

_L = 128
_CHUNK = 13
_D = 32


def _emb_kernel(n_rows_total, table_hbm, idx_hbm, out_hbm,
                idx_v, rows_a, rows_b, sem_ga, sem_gb, sem_wa, sem_wb):
    wid = lax.axis_index("s") * 2 + lax.axis_index("c")
    rows_per_w = n_rows_total // 32
    base = wid * rows_per_w
    pltpu.sync_copy(idx_hbm.at[pl.ds(base, rows_per_w)], idx_v)

    n_chunks = rows_per_w // _CHUNK
    n_pairs = n_chunks // 2

    def g_start(buf, sem, c):
        row0 = c * _CHUNK
        for j in range(_CHUNK):
            pltpu.async_copy(table_hbm.at[idx_v.at[row0 + j]], buf.at[j], sem)

    def g_drain(buf, sem):
        pltpu.make_async_copy(out_hbm.at[pl.ds(0, _CHUNK)], buf, sem).wait()

    g_start(rows_a, sem_ga, 0)
    g_start(rows_b, sem_gb, 1)

    def body(p, carry):
        c0 = p * 2
        g_drain(rows_a, sem_ga)
        wa = pltpu.async_copy(
            rows_a, out_hbm.at[pl.ds(base + c0 * _CHUNK, _CHUNK)], sem_wa)
        g_drain(rows_b, sem_gb)
        wb = pltpu.async_copy(
            rows_b, out_hbm.at[pl.ds(base + (c0 + 1) * _CHUNK, _CHUNK)], sem_wb)

        wa.wait()

        @pl.when(p < n_pairs - 1)
        def _():
            g_start(rows_a, sem_ga, c0 + 2)

        wb.wait()

        @pl.when(p < n_pairs - 1)
        def _():
            g_start(rows_b, sem_gb, c0 + 3)

        return carry

    lax.fori_loop(0, n_pairs, body, 0)


def kernel(x, emb_weight):
    n, m = x.shape
    total = n * m
    n_rows = total // _L
    idx2d = x.reshape(n_rows, _L).astype(jnp.int32)

    mesh = plsc.VectorSubcoreMesh(core_axis_name="c", subcore_axis_name="s")
    rows_per_w = n_rows // 32

    k = functools.partial(
        pl.kernel,
        mesh=mesh,
        out_type=jax.ShapeDtypeStruct((n_rows, _L, _D), jnp.float32),
        scratch_types=[
            pltpu.VMEM((rows_per_w, _L), jnp.int32),
            pltpu.VMEM((_CHUNK, _L, _D), jnp.float32),
            pltpu.VMEM((_CHUNK, _L, _D), jnp.float32),
            pltpu.SemaphoreType.DMA,
            pltpu.SemaphoreType.DMA,
            pltpu.SemaphoreType.DMA,
            pltpu.SemaphoreType.DMA,
        ],
        compiler_params=pltpu.CompilerParams(use_tc_tiling_on_sc=False),
    )(functools.partial(_emb_kernel, n_rows))

    out = k(emb_weight, idx2d)
    return out.reshape(n, m, _D)
